# baseline (speedup 1.0000x reference)
import functools

import jax
import jax.numpy as jnp
from jax import lax
from jax.experimental import pallas as pl
from jax.experimental.pallas import tpu as pltpu
from jax.experimental.pallas import tpu_sc as plsc

B = 4096
L = 200
NUM_USER = 1000000
NUM_MOVIE = 100000
D_USER = 64
D_SMALL = 16
D_MOVIE = 64
D_FEAT = D_USER + 3 * D_SMALL

_INFO = plsc.get_sparse_core_info()
NC = _INFO.num_cores
NS = _INFO.num_subcores
NW = NC * NS

BPW = B // NW
N_CHUNKS = L
N_BLOCKS = N_CHUNKS // 4
N_TILES = L // 8
ULAST = ((NUM_USER - 1) // 128) * 128


def _body(user_id, sex, age, occupation, target_item, seq_t, user_t,
          sex_table, age_table, occupation_table, mt,
          out2d, tgt_t, feat_t,
          uid_v, sid_v, aid_v, oid_v, tid_v,
          sexv, agev, occv, idx_v, ring, trans, ubuf, feat_v,
          gsem, wsem, isem, usem):
    wid = lax.axis_index("s") * NC + lax.axis_index("c")
    b0 = pl.multiple_of(wid * BPW, 128)
    iota = lax.iota(jnp.int32, 16)

    def extract(rs, ts):
        rsv = jnp.full((16,), rs, jnp.int32)

        def dbody(d, carry):
            dv = jnp.full((16,), d, jnp.int32)
            for jg in range(8):
                jv = jg * 16 + iota
                vals = plsc.load_gather(ring, [rsv, jv, dv])
                trans[ts, d, pl.ds(jg * 16, 16)] = vals
            return carry

        lax.fori_loop(0, D_MOVIE, dbody, None)

    pltpu.sync_copy(user_id.at[pl.ds(b0, BPW)], uid_v)
    pltpu.sync_copy(sex.at[pl.ds(b0, BPW)], sid_v)
    pltpu.sync_copy(age.at[pl.ds(b0, BPW)], aid_v)
    pltpu.sync_copy(occupation.at[pl.ds(b0, BPW)], oid_v)
    pltpu.sync_copy(target_item.at[pl.ds(b0, BPW)], tid_v)
    pltpu.sync_copy(sex_table, sexv)
    pltpu.sync_copy(age_table, agev)
    pltpu.sync_copy(occupation_table, occv)

    for jg in range(8):
        jsl = pl.ds(jg * 16, 16)
        sv = sid_v[jsl]
        av = aid_v[jsl]
        ov = oid_v[jsl]

        def kbody(k, carry, sv=sv, av=av, ov=ov, jsl=jsl):
            kv = jnp.full((16,), k, jnp.int32)
            feat_v[D_USER + k, jsl] = plsc.load_gather(sexv, [sv, kv])
            feat_v[D_USER + D_SMALL + k, jsl] = plsc.load_gather(agev, [av, kv])
            feat_v[D_USER + 2 * D_SMALL + k, jsl] = plsc.load_gather(occv, [ov, kv])
            return carry

        lax.fori_loop(0, D_SMALL, kbody, None)

    def ucol(j):
        g = (j >> 4) << 4
        uv = uid_v[pl.ds(g, 16)]
        r = lax.reduce_max(jnp.where(iota == j - g, uv, 0), (0,))
        c0 = pl.multiple_of(jnp.minimum((r >> 7) << 7, ULAST), 128)
        return c0, r - c0

    c0_first, _ = ucol(0)
    pltpu.async_copy(user_t.at[:, pl.ds(c0_first, 128)], ubuf.at[0], usem)

    def ubody(j, carry):
        slot = j % 2
        _, col = ucol(j)
        pltpu.make_async_copy(user_t.at[:, pl.ds(0, 128)], ubuf.at[slot], usem).wait()

        @pl.when(j < BPW - 1)
        def _next():
            c0n, _ = ucol(j + 1)
            pltpu.async_copy(user_t.at[:, pl.ds(c0n, 128)], ubuf.at[1 - slot], usem)

        slotv = jnp.full((16,), slot, jnp.int32)
        colv = jnp.full((16,), col, jnp.int32)
        jv16 = jnp.full((16,), j, jnp.int32)
        for dg in range(4):
            dv = dg * 16 + iota
            vals = plsc.load_gather(ubuf, [slotv, dv, colv])
            plsc.store_scatter(feat_v, [dv, jv16], vals)
        return carry

    lax.fori_loop(0, BPW, ubody, None)
    pltpu.sync_copy(feat_v, feat_t.at[:, pl.ds(b0, BPW)])

    pltpu.async_copy(mt.at[tid_v], ring.at[0], gsem).wait()
    extract(0, 0)
    pltpu.sync_copy(trans.at[0], tgt_t.at[:, pl.ds(b0, BPW)])

    pltpu.sync_copy(seq_t.at[pl.ds(0, 8), pl.ds(b0, BPW)], idx_v.at[0])
    for c in range(4):
        pltpu.async_copy(mt.at[idx_v.at[0, c]], ring.at[c], gsem)

    def out_at(t):
        row = pl.multiple_of(t * D_MOVIE, D_MOVIE)
        return out2d.at[pl.ds(row, D_MOVIE), pl.ds(b0, BPW)]

    def block(i, carry):
        @pl.when((i % 2 == 0) & (i < 2 * (N_TILES - 1)))
        def _tile_fire():
            tn = i // 2 + 1
            trow = pl.multiple_of(tn * 8, 8)
            pltpu.async_copy(seq_t.at[pl.ds(trow, 8), pl.ds(b0, BPW)],
                             idx_v.at[tn % 2], isem)

        @pl.when((i % 2 == 1) & (i < 2 * (N_TILES - 1)))
        def _tile_drain():
            pltpu.make_async_copy(seq_t.at[pl.ds(0, 8), pl.ds(b0, BPW)],
                                  idx_v.at[0], isem).wait()

        for c in range(4):
            t = 4 * i + c
            pltpu.make_async_copy(mt.at[idx_v.at[0, 0]], ring.at[c], gsem).wait()

            @pl.when(t >= 2)
            def _wdrain(t=t, c=c):
                pltpu.make_async_copy(trans.at[c % 2], out_at(t - 2), wsem).wait()

            extract(c, c % 2)
            pltpu.async_copy(trans.at[c % 2], out_at(t), wsem)

            @pl.when(t + 4 < N_CHUNKS)
            def _refill(t=t, c=c):
                t4 = t + 4
                pltpu.async_copy(mt.at[idx_v.at[(t4 // 8) % 2, t4 % 8]],
                                 ring.at[c], gsem)
        return carry

    lax.fori_loop(0, N_BLOCKS, block, None)
    pltpu.make_async_copy(trans.at[0], out_at(N_CHUNKS - 2), wsem).wait()
    pltpu.make_async_copy(trans.at[1], out_at(N_CHUNKS - 1), wsem).wait()


@jax.jit
def _run(user_id, sex, age, occupation, target_item, seq_t, user_t,
         sex_table, age_table, occupation_table, mt):
    mesh = plsc.VectorSubcoreMesh(core_axis_name="c", subcore_axis_name="s")
    k = functools.partial(
        pl.kernel,
        mesh=mesh,
        compiler_params=pltpu.CompilerParams(use_tc_tiling_on_sc=True,
                                             needs_layout_passes=False),
        out_type=[
            jax.ShapeDtypeStruct((L * D_MOVIE, B), jnp.float32),
            jax.ShapeDtypeStruct((D_MOVIE, B), jnp.float32),
            jax.ShapeDtypeStruct((D_FEAT, B), jnp.float32),
        ],
        scratch_types=[
            pltpu.VMEM((BPW,), jnp.int32),
            pltpu.VMEM((BPW,), jnp.int32),
            pltpu.VMEM((BPW,), jnp.int32),
            pltpu.VMEM((BPW,), jnp.int32),
            pltpu.VMEM((BPW,), jnp.int32),
            pltpu.VMEM((2, D_SMALL), jnp.float32),
            pltpu.VMEM((7, D_SMALL), jnp.float32),
            pltpu.VMEM((21, D_SMALL), jnp.float32),
            pltpu.VMEM((2, 8, 128), jnp.int32),
            pltpu.VMEM((4, 128, 128), jnp.float32),
            pltpu.VMEM((2, D_MOVIE, 128), jnp.float32),
            pltpu.VMEM((2, D_USER, 128), jnp.float32),
            pltpu.VMEM((D_FEAT, 128), jnp.float32),
            pltpu.SemaphoreType.DMA,
            pltpu.SemaphoreType.DMA,
            pltpu.SemaphoreType.DMA,
            pltpu.SemaphoreType.DMA,
        ],
    )(_body)
    return k(user_id, sex, age, occupation, target_item, seq_t, user_t,
             sex_table, age_table, occupation_table, mt)


def kernel(user_id, sex, age, occupation, seq_item, target_item,
           user_table, sex_table, age_table, occupation_table, movie_table):
    seq_t = jnp.transpose(seq_item.astype(jnp.int32))
    user_t = jnp.transpose(user_table)
    mt = jnp.pad(movie_table, ((0, 0), (0, 64)))
    out2d, tgt_t, feat_t = _run(
        user_id.astype(jnp.int32), sex.astype(jnp.int32), age.astype(jnp.int32),
        occupation.astype(jnp.int32), target_item.astype(jnp.int32),
        seq_t, user_t, sex_table, age_table, occupation_table, mt)
    seq_out = jnp.transpose(out2d.reshape(L, D_MOVIE, B), (2, 0, 1))
    tgt = jnp.transpose(tgt_t)
    feat = jnp.transpose(feat_t)
    return (feat, seq_out, tgt)


# speedup vs baseline: 1.1928x; 1.1928x over previous
import functools

import jax
import jax.numpy as jnp
from jax import lax
from jax.experimental import pallas as pl
from jax.experimental.pallas import tpu as pltpu
from jax.experimental.pallas import tpu_sc as plsc

B = 4096
L = 200
NUM_USER = 1000000
NUM_MOVIE = 100000
D_USER = 64
D_SMALL = 16
D_MOVIE = 64
D_FEAT = D_USER + 3 * D_SMALL

_INFO = plsc.get_sparse_core_info()
NC = _INFO.num_cores
NS = _INFO.num_subcores
NW = NC * NS

BPW = B // NW
N_CHUNKS = L
N_BLOCKS = N_CHUNKS // 4
N_TILES = L // 8
ULAST = ((NUM_USER - 1) // 128) * 128


def _body(user_id, sex, age, occupation, target_item, seq_t, user_t,
          sex_table, age_table, occupation_table, mt,
          out2d, tgt_t, feat_t,
          uid_v, sid_v, aid_v, oid_v, tid_v,
          sexv, agev, occv, idx_v, ring, trans, ubuf, feat_v,
          gsem, wsem, isem, usem):
    wid = lax.axis_index("s") * NC + lax.axis_index("c")
    b0 = pl.multiple_of(wid * BPW, 128)
    iota = lax.iota(jnp.int32, 16)

    dcols = [dg * 16 + iota for dg in range(4)]

    def extract(rs, tbuf):
        def jbody(j, carry):
            jb = jnp.full((16,), j, jnp.int32)
            for dg in range(4):
                vals = ring[rs, j, pl.ds(dg * 16, 16)]
                plsc.store_scatter(tbuf, [dcols[dg], jb], vals)
            return carry

        lax.fori_loop(0, 128, jbody, None)

    pltpu.sync_copy(user_id.at[pl.ds(b0, BPW)], uid_v)
    pltpu.sync_copy(sex.at[pl.ds(b0, BPW)], sid_v)
    pltpu.sync_copy(age.at[pl.ds(b0, BPW)], aid_v)
    pltpu.sync_copy(occupation.at[pl.ds(b0, BPW)], oid_v)
    pltpu.sync_copy(target_item.at[pl.ds(b0, BPW)], tid_v)
    pltpu.sync_copy(sex_table, sexv)
    pltpu.sync_copy(age_table, agev)
    pltpu.sync_copy(occupation_table, occv)

    for jg in range(8):
        jsl = pl.ds(jg * 16, 16)
        sv = sid_v[jsl]
        av = aid_v[jsl]
        ov = oid_v[jsl]

        def kbody(k, carry, sv=sv, av=av, ov=ov, jsl=jsl):
            kv = jnp.full((16,), k, jnp.int32)
            feat_v[D_USER + k, jsl] = plsc.load_gather(sexv, [sv, kv])
            feat_v[D_USER + D_SMALL + k, jsl] = plsc.load_gather(agev, [av, kv])
            feat_v[D_USER + 2 * D_SMALL + k, jsl] = plsc.load_gather(occv, [ov, kv])
            return carry

        lax.fori_loop(0, D_SMALL, kbody, None)

    def ucol(j):
        g = (j >> 4) << 4
        uv = uid_v[pl.ds(g, 16)]
        r = lax.reduce_max(jnp.where(iota == j - g, uv, 0), (0,))
        c0 = pl.multiple_of(jnp.minimum((r >> 7) << 7, ULAST), 128)
        return c0, r - c0

    c0_first, _ = ucol(0)
    pltpu.async_copy(user_t.at[:, pl.ds(c0_first, 128)], ubuf.at[0], usem)

    def ubody(j, carry):
        slot = j % 2
        _, col = ucol(j)
        pltpu.make_async_copy(user_t.at[:, pl.ds(0, 128)], ubuf.at[slot], usem).wait()

        @pl.when(j < BPW - 1)
        def _next():
            c0n, _ = ucol(j + 1)
            pltpu.async_copy(user_t.at[:, pl.ds(c0n, 128)], ubuf.at[1 - slot], usem)

        slotv = jnp.full((16,), slot, jnp.int32)
        colv = jnp.full((16,), col, jnp.int32)
        jv16 = jnp.full((16,), j, jnp.int32)
        for dg in range(4):
            dv = dg * 16 + iota
            vals = plsc.load_gather(ubuf, [slotv, dv, colv])
            plsc.store_scatter(feat_v, [dv, jv16], vals)
        return carry

    lax.fori_loop(0, BPW, ubody, None)
    pltpu.sync_copy(feat_v, feat_t.at[:, pl.ds(b0, BPW)])

    pltpu.async_copy(mt.at[tid_v], ring.at[0], gsem).wait()
    extract(0, trans.at[0])
    pltpu.sync_copy(trans.at[0], tgt_t.at[:, pl.ds(b0, BPW)])

    pltpu.sync_copy(seq_t.at[pl.ds(0, 8), pl.ds(b0, BPW)], idx_v.at[0])
    for c in range(4):
        pltpu.async_copy(mt.at[idx_v.at[0, c]], ring.at[c], gsem)

    def out_at(t):
        row = pl.multiple_of(t * D_MOVIE, D_MOVIE)
        return out2d.at[pl.ds(row, D_MOVIE), pl.ds(b0, BPW)]

    def block(i, carry):
        @pl.when((i % 2 == 0) & (i < 2 * (N_TILES - 1)))
        def _tile_fire():
            tn = i // 2 + 1
            trow = pl.multiple_of(tn * 8, 8)
            pltpu.async_copy(seq_t.at[pl.ds(trow, 8), pl.ds(b0, BPW)],
                             idx_v.at[tn % 2], isem)

        @pl.when((i % 2 == 1) & (i < 2 * (N_TILES - 1)))
        def _tile_drain():
            pltpu.make_async_copy(seq_t.at[pl.ds(0, 8), pl.ds(b0, BPW)],
                                  idx_v.at[0], isem).wait()

        for c in range(4):
            t = 4 * i + c
            pltpu.make_async_copy(mt.at[idx_v.at[0, 0]], ring.at[c], gsem).wait()

            @pl.when(t >= 2)
            def _wdrain(t=t, c=c):
                pltpu.make_async_copy(trans.at[c % 2], out_at(t - 2), wsem).wait()

            extract(c, trans.at[c % 2])
            pltpu.async_copy(trans.at[c % 2], out_at(t), wsem)

            @pl.when(t + 4 < N_CHUNKS)
            def _refill(t=t, c=c):
                t4 = t + 4
                pltpu.async_copy(mt.at[idx_v.at[(t4 // 8) % 2, t4 % 8]],
                                 ring.at[c], gsem)
        return carry

    lax.fori_loop(0, N_BLOCKS, block, None)
    pltpu.make_async_copy(trans.at[0], out_at(N_CHUNKS - 2), wsem).wait()
    pltpu.make_async_copy(trans.at[1], out_at(N_CHUNKS - 1), wsem).wait()


@jax.jit
def _run(user_id, sex, age, occupation, target_item, seq_t, user_t,
         sex_table, age_table, occupation_table, mt):
    mesh = plsc.VectorSubcoreMesh(core_axis_name="c", subcore_axis_name="s")
    k = functools.partial(
        pl.kernel,
        mesh=mesh,
        compiler_params=pltpu.CompilerParams(use_tc_tiling_on_sc=True,
                                             needs_layout_passes=False),
        out_type=[
            jax.ShapeDtypeStruct((L * D_MOVIE, B), jnp.float32),
            jax.ShapeDtypeStruct((D_MOVIE, B), jnp.float32),
            jax.ShapeDtypeStruct((D_FEAT, B), jnp.float32),
        ],
        scratch_types=[
            pltpu.VMEM((BPW,), jnp.int32),
            pltpu.VMEM((BPW,), jnp.int32),
            pltpu.VMEM((BPW,), jnp.int32),
            pltpu.VMEM((BPW,), jnp.int32),
            pltpu.VMEM((BPW,), jnp.int32),
            pltpu.VMEM((2, D_SMALL), jnp.float32),
            pltpu.VMEM((7, D_SMALL), jnp.float32),
            pltpu.VMEM((21, D_SMALL), jnp.float32),
            pltpu.VMEM((2, 8, 128), jnp.int32),
            pltpu.VMEM((4, 128, 128), jnp.float32),
            pltpu.VMEM((2, D_MOVIE, 128), jnp.float32),
            pltpu.VMEM((2, D_USER, 128), jnp.float32),
            pltpu.VMEM((D_FEAT, 128), jnp.float32),
            pltpu.SemaphoreType.DMA,
            pltpu.SemaphoreType.DMA,
            pltpu.SemaphoreType.DMA,
            pltpu.SemaphoreType.DMA,
        ],
    )(_body)
    return k(user_id, sex, age, occupation, target_item, seq_t, user_t,
             sex_table, age_table, occupation_table, mt)


def kernel(user_id, sex, age, occupation, seq_item, target_item,
           user_table, sex_table, age_table, occupation_table, movie_table):
    seq_t = jnp.transpose(seq_item.astype(jnp.int32))
    user_t = jnp.transpose(user_table)
    mt = jnp.pad(movie_table, ((0, 0), (0, 64)))
    out2d, tgt_t, feat_t = _run(
        user_id.astype(jnp.int32), sex.astype(jnp.int32), age.astype(jnp.int32),
        occupation.astype(jnp.int32), target_item.astype(jnp.int32),
        seq_t, user_t, sex_table, age_table, occupation_table, mt)
    seq_out = jnp.transpose(out2d.reshape(L, D_MOVIE, B), (2, 0, 1))
    tgt = jnp.transpose(tgt_t)
    feat = jnp.transpose(feat_t)
    return (feat, seq_out, tgt)


# speedup vs baseline: 1.2104x; 1.0147x over previous
import functools

import jax
import jax.numpy as jnp
from jax import lax
from jax.experimental import pallas as pl
from jax.experimental.pallas import tpu as pltpu
from jax.experimental.pallas import tpu_sc as plsc

B = 4096
L = 200
NUM_USER = 1000000
NUM_MOVIE = 100000
D_USER = 64
D_SMALL = 16
D_MOVIE = 64
D_FEAT = D_USER + 3 * D_SMALL

_INFO = plsc.get_sparse_core_info()
NC = _INFO.num_cores
NS = _INFO.num_subcores
NW = NC * NS

BPW = B // NW
N_CHUNKS = L
N_BLOCKS = N_CHUNKS // 4
N_TILES = L // 8
ULAST = ((NUM_USER - 1) // 128) * 128


def _body(user_id, sex, age, occupation, target_item, seq_t, user_t,
          sex_table, age_table, occupation_table, mt,
          out2d, tgt_t, feat_t,
          uid_v, sid_v, aid_v, oid_v, tid_v,
          sexv, agev, occv, idx_v, ring, trans, ubuf, feat_v,
          gsem, wsem, isem, usem):
    wid = lax.axis_index("s") * NC + lax.axis_index("c")
    b0 = pl.multiple_of(wid * BPW, 128)
    iota = lax.iota(jnp.int32, 16)

    dcols = [dg * 16 + iota for dg in range(4)]

    def extract(rs, tbuf):
        def jbody(i2, jb):
            jb1 = jb + 1
            va = [ring[rs, 2 * i2, pl.ds(dg * 16, 16)] for dg in range(4)]
            vb = [ring[rs, 2 * i2 + 1, pl.ds(dg * 16, 16)] for dg in range(4)]
            for dg in range(4):
                plsc.store_scatter(tbuf, [dcols[dg], jb], va[dg])
            for dg in range(4):
                plsc.store_scatter(tbuf, [dcols[dg], jb1], vb[dg])
            return jb + 2

        lax.fori_loop(0, 64, jbody, jnp.zeros((16,), jnp.int32))

    pltpu.sync_copy(user_id.at[pl.ds(b0, BPW)], uid_v)
    pltpu.sync_copy(sex.at[pl.ds(b0, BPW)], sid_v)
    pltpu.sync_copy(age.at[pl.ds(b0, BPW)], aid_v)
    pltpu.sync_copy(occupation.at[pl.ds(b0, BPW)], oid_v)
    pltpu.sync_copy(target_item.at[pl.ds(b0, BPW)], tid_v)
    pltpu.sync_copy(sex_table, sexv)
    pltpu.sync_copy(age_table, agev)
    pltpu.sync_copy(occupation_table, occv)

    for jg in range(8):
        jsl = pl.ds(jg * 16, 16)
        sv = sid_v[jsl]
        av = aid_v[jsl]
        ov = oid_v[jsl]

        def kbody(k, carry, sv=sv, av=av, ov=ov, jsl=jsl):
            kv = jnp.full((16,), k, jnp.int32)
            feat_v[D_USER + k, jsl] = plsc.load_gather(sexv, [sv, kv])
            feat_v[D_USER + D_SMALL + k, jsl] = plsc.load_gather(agev, [av, kv])
            feat_v[D_USER + 2 * D_SMALL + k, jsl] = plsc.load_gather(occv, [ov, kv])
            return carry

        lax.fori_loop(0, D_SMALL, kbody, None)

    def ucol(j):
        g = (j >> 4) << 4
        uv = uid_v[pl.ds(g, 16)]
        r = lax.reduce_max(jnp.where(iota == j - g, uv, 0), (0,))
        c0 = pl.multiple_of(jnp.minimum((r >> 7) << 7, ULAST), 128)
        return c0, r - c0

    c0_first, _ = ucol(0)
    pltpu.async_copy(user_t.at[:, pl.ds(c0_first, 128)], ubuf.at[0], usem)

    def ubody(j, carry):
        slot = j % 2
        _, col = ucol(j)
        pltpu.make_async_copy(user_t.at[:, pl.ds(0, 128)], ubuf.at[slot], usem).wait()

        @pl.when(j < BPW - 1)
        def _next():
            c0n, _ = ucol(j + 1)
            pltpu.async_copy(user_t.at[:, pl.ds(c0n, 128)], ubuf.at[1 - slot], usem)

        slotv = jnp.full((16,), slot, jnp.int32)
        colv = jnp.full((16,), col, jnp.int32)
        jv16 = jnp.full((16,), j, jnp.int32)
        for dg in range(4):
            dv = dg * 16 + iota
            vals = plsc.load_gather(ubuf, [slotv, dv, colv])
            plsc.store_scatter(feat_v, [dv, jv16], vals)
        return carry

    lax.fori_loop(0, BPW, ubody, None)
    pltpu.sync_copy(feat_v, feat_t.at[:, pl.ds(b0, BPW)])

    pltpu.async_copy(mt.at[tid_v], ring.at[0], gsem).wait()
    extract(0, trans.at[0])
    pltpu.sync_copy(trans.at[0], tgt_t.at[:, pl.ds(b0, BPW)])

    pltpu.sync_copy(seq_t.at[pl.ds(0, 8), pl.ds(b0, BPW)], idx_v.at[0])
    for c in range(4):
        pltpu.async_copy(mt.at[idx_v.at[0, c]], ring.at[c], gsem)

    def out_at(t):
        row = pl.multiple_of(t * D_MOVIE, D_MOVIE)
        return out2d.at[pl.ds(row, D_MOVIE), pl.ds(b0, BPW)]

    def block(i, carry):
        @pl.when((i % 2 == 0) & (i < 2 * (N_TILES - 1)))
        def _tile_fire():
            tn = i // 2 + 1
            trow = pl.multiple_of(tn * 8, 8)
            pltpu.async_copy(seq_t.at[pl.ds(trow, 8), pl.ds(b0, BPW)],
                             idx_v.at[tn % 2], isem)

        @pl.when((i % 2 == 1) & (i < 2 * (N_TILES - 1)))
        def _tile_drain():
            pltpu.make_async_copy(seq_t.at[pl.ds(0, 8), pl.ds(b0, BPW)],
                                  idx_v.at[0], isem).wait()

        for c in range(4):
            t = 4 * i + c
            pltpu.make_async_copy(mt.at[idx_v.at[0, 0]], ring.at[c], gsem).wait()

            @pl.when(t >= 2)
            def _wdrain(t=t, c=c):
                pltpu.make_async_copy(trans.at[c % 2], out_at(t - 2), wsem).wait()

            extract(c, trans.at[c % 2])
            pltpu.async_copy(trans.at[c % 2], out_at(t), wsem)

            @pl.when(t + 4 < N_CHUNKS)
            def _refill(t=t, c=c):
                t4 = t + 4
                pltpu.async_copy(mt.at[idx_v.at[(t4 // 8) % 2, t4 % 8]],
                                 ring.at[c], gsem)
        return carry

    lax.fori_loop(0, N_BLOCKS, block, None)
    pltpu.make_async_copy(trans.at[0], out_at(N_CHUNKS - 2), wsem).wait()
    pltpu.make_async_copy(trans.at[1], out_at(N_CHUNKS - 1), wsem).wait()


@jax.jit
def _run(user_id, sex, age, occupation, target_item, seq_t, user_t,
         sex_table, age_table, occupation_table, mt):
    mesh = plsc.VectorSubcoreMesh(core_axis_name="c", subcore_axis_name="s")
    k = functools.partial(
        pl.kernel,
        mesh=mesh,
        compiler_params=pltpu.CompilerParams(use_tc_tiling_on_sc=True,
                                             needs_layout_passes=False),
        out_type=[
            jax.ShapeDtypeStruct((L * D_MOVIE, B), jnp.float32),
            jax.ShapeDtypeStruct((D_MOVIE, B), jnp.float32),
            jax.ShapeDtypeStruct((D_FEAT, B), jnp.float32),
        ],
        scratch_types=[
            pltpu.VMEM((BPW,), jnp.int32),
            pltpu.VMEM((BPW,), jnp.int32),
            pltpu.VMEM((BPW,), jnp.int32),
            pltpu.VMEM((BPW,), jnp.int32),
            pltpu.VMEM((BPW,), jnp.int32),
            pltpu.VMEM((2, D_SMALL), jnp.float32),
            pltpu.VMEM((7, D_SMALL), jnp.float32),
            pltpu.VMEM((21, D_SMALL), jnp.float32),
            pltpu.VMEM((2, 8, 128), jnp.int32),
            pltpu.VMEM((4, 128, 128), jnp.float32),
            pltpu.VMEM((2, D_MOVIE, 128), jnp.float32),
            pltpu.VMEM((2, D_USER, 128), jnp.float32),
            pltpu.VMEM((D_FEAT, 128), jnp.float32),
            pltpu.SemaphoreType.DMA,
            pltpu.SemaphoreType.DMA,
            pltpu.SemaphoreType.DMA,
            pltpu.SemaphoreType.DMA,
        ],
    )(_body)
    return k(user_id, sex, age, occupation, target_item, seq_t, user_t,
             sex_table, age_table, occupation_table, mt)


def kernel(user_id, sex, age, occupation, seq_item, target_item,
           user_table, sex_table, age_table, occupation_table, movie_table):
    seq_t = jnp.transpose(seq_item.astype(jnp.int32))
    user_t = jnp.transpose(user_table)
    mt = jnp.pad(movie_table, ((0, 0), (0, 64)))
    out2d, tgt_t, feat_t = _run(
        user_id.astype(jnp.int32), sex.astype(jnp.int32), age.astype(jnp.int32),
        occupation.astype(jnp.int32), target_item.astype(jnp.int32),
        seq_t, user_t, sex_table, age_table, occupation_table, mt)
    seq_out = jnp.transpose(out2d.reshape(L, D_MOVIE, B), (2, 0, 1))
    tgt = jnp.transpose(tgt_t)
    feat = jnp.transpose(feat_t)
    return (feat, seq_out, tgt)
